# baseline (speedup 1.0000x reference)
import functools

import jax
import jax.numpy as jnp
from jax import lax
from jax.experimental import pallas as pl
from jax.experimental.pallas import tpu as pltpu
from jax.experimental.pallas import tpu_sc as plsc

VOCAB = 1000000
B = 16384
L = 50
D = 32
C = 16

NC = 2
NS = 16
NW = NC * NS
RPW = B // NW
CHUNK = 128
NCH = RPW // CHUNK


def _sc_bag_sum(idx_t, table):
    mesh = plsc.VectorSubcoreMesh(
        core_axis_name="c", subcore_axis_name="s", num_cores=NC, num_subcores=NS
    )

    @functools.partial(
        pl.kernel,
        mesh=mesh,
        out_type=jax.ShapeDtypeStruct((B, D), jnp.float32),
        scratch_types=[
            pltpu.VMEM((L, RPW), jnp.int32),
            pltpu.VMEM((RPW, D), jnp.float32),
            pltpu.SemaphoreType.DMA,
        ],
        compiler_params=pltpu.CompilerParams(use_tc_tiling_on_sc=False),
    )
    def k(idx_hbm, tbl_hbm, out_hbm, idx_v, acc_v, sem):
        wid = lax.axis_index("s") * NC + lax.axis_index("c")
        base = wid * RPW
        pltpu.sync_copy(idx_hbm.at[:, pl.ds(base, RPW)], idx_v)

        zero = jnp.zeros((D,), jnp.float32)

        def zero_row(r, _):
            acc_v[r] = zero
            return 0

        lax.fori_loop(0, RPW, zero_row, 0)

        for c in range(NCH):
            sl = pl.ds(c * CHUNK, CHUNK)
            dst = acc_v.at[pl.ds(c * CHUNK, CHUNK)]

            def fire(j, _):
                pltpu.async_copy(
                    tbl_hbm.at[idx_v.at[j, sl]], dst, sem, add=True
                )
                return 0

            lax.fori_loop(0, L, fire, 0)

        drain = pltpu.make_async_copy(
            tbl_hbm.at[pl.ds(0, CHUNK)], acc_v.at[pl.ds(0, CHUNK)], sem
        )

        def drain_one(i, _):
            drain.wait()
            return 0

        lax.fori_loop(0, NCH * L, drain_one, 0)
        pltpu.sync_copy(acc_v, out_hbm.at[pl.ds(base, RPW)])

    return k(idx_t, table)


def _tc_linear(x, w_t, bias):
    BB = 4096

    def body(x_ref, w_ref, b_ref, o_ref):
        o_ref[...] = (
            jnp.dot(x_ref[...], w_ref[...], preferred_element_type=jnp.float32)
            + b_ref[...]
        )

    return pl.pallas_call(
        body,
        grid=(B // BB,),
        in_specs=[
            pl.BlockSpec((BB, D), lambda i: (i, 0)),
            pl.BlockSpec((D, C), lambda i: (0, 0)),
            pl.BlockSpec((1, C), lambda i: (0, 0)),
        ],
        out_specs=pl.BlockSpec((BB, C), lambda i: (i, 0)),
        out_shape=jax.ShapeDtypeStruct((B, C), jnp.float32),
    )(x, w_t, bias)


def kernel(text, emb_weight, fc_weight, fc_bias):
    idx_t = jnp.swapaxes(text.astype(jnp.int32), 0, 1)
    sums = _sc_bag_sum(idx_t, emb_weight)
    w_t = jnp.swapaxes(fc_weight, 0, 1) * (1.0 / L)
    return _tc_linear(sums, w_t, fc_bias.reshape(1, C))
